# baseline (speedup 1.0000x reference)
import functools

import jax
import jax.numpy as jnp
from jax import lax
from jax.experimental import pallas as pl
from jax.experimental.pallas import tpu as pltpu
from jax.experimental.pallas import tpu_sc as plsc

NC = 2
NS = 16
L = 16
NW = NC * NS

B = 16384
D = 16
BPW = B // NW
NCHUNK = 4
CHUNK = BPW // NCHUNK


def _body(idx0_hbm, idx1_hbm, f0_hbm, f1_hbm, out_hbm,
          idx0_v, idx1_v, rows0_v, rows1_v, out_v, sem0, sem1):
  wid = lax.axis_index("s") * NC + lax.axis_index("c")
  base = wid * BPW

  pltpu.sync_copy(idx0_hbm.at[wid], idx0_v)
  pltpu.sync_copy(idx1_hbm.at[wid], idx1_v)

  copies = []
  for j in range(NCHUNK):
    copies.append(pltpu.async_copy(
        f0_hbm.at[idx0_v.at[j]],
        rows0_v.at[pl.ds(j * CHUNK, CHUNK), :], sem0))
    copies.append(pltpu.async_copy(
        f1_hbm.at[idx1_v.at[j]],
        rows1_v.at[pl.ds(j * CHUNK, CHUNK), :], sem1))
  for c in copies:
    c.wait()

  iota = lax.broadcasted_iota(jnp.int32, (L,), 0)

  def chunk16(c, carry):
    acc = jnp.zeros((L,), jnp.float32)
    for i in range(L):
      b = c * L + i
      s = jnp.sum(rows0_v[b, :] * rows1_v[b, :])
      acc = jnp.where(iota == i, s, acc)
    plsc.store_scatter(out_v, [c * L + iota], acc)
    return carry

  lax.fori_loop(0, BPW // L, chunk16, 0)

  pltpu.sync_copy(out_v, out_hbm.at[pl.ds(base, BPW)])


def kernel(idx0, idx1, factor0, factor1):
  mesh = plsc.VectorSubcoreMesh(
      core_axis_name="c", subcore_axis_name="s",
      num_cores=NC, num_subcores=NS)
  run = pl.kernel(
      _body,
      out_type=jax.ShapeDtypeStruct((B,), jnp.float32),
      mesh=mesh,
      scratch_types=[
          pltpu.VMEM((NCHUNK, CHUNK), jnp.int32),
          pltpu.VMEM((NCHUNK, CHUNK), jnp.int32),
          pltpu.VMEM((BPW, D), jnp.float32),
          pltpu.VMEM((BPW, D), jnp.float32),
          pltpu.VMEM((BPW,), jnp.float32),
          pltpu.SemaphoreType.DMA,
          pltpu.SemaphoreType.DMA,
      ],
      compiler_params=pltpu.CompilerParams(
          needs_layout_passes=False, use_tc_tiling_on_sc=False),
  )
  return run(idx0.reshape(NW, NCHUNK, CHUNK),
             idx1.reshape(NW, NCHUNK, CHUNK),
             factor0, factor1)
